# baseline (speedup 1.0000x reference)
import functools

import jax
import jax.numpy as jnp
from jax import lax
from jax.experimental import pallas as pl
from jax.experimental.pallas import tpu as pltpu
from jax.experimental.pallas import tpu_sc as plsc

ROWS = 1024
VOCAB = 100000
NLAB = 100
LPAD = 128
LBLK = 32
RBLK = 128
NQ = 4
NH = RBLK // 16
LN2 = 0.6931471805599453
NEG = -3.4e38


def _log_lanes(sv):
    bits = lax.bitcast_convert_type(sv, jnp.int32)
    e = lax.shift_right_logical(bits, 23) - 127
    mb = lax.bitwise_or(lax.bitwise_and(bits, 0x007FFFFF), 0x3F800000)
    mf = lax.bitcast_convert_type(mb, jnp.float32)
    big = mf > 1.4142135
    mf = jnp.where(big, mf * 0.5, mf)
    e = jnp.where(big, e + 1, e)
    t = (mf - 1.0) / (mf + 1.0)
    t2 = t * t
    poly = 1.0 + t2 * (
        0.3333333333 + t2 * (0.2 + t2 * (0.1428571429 + t2 * 0.1111111111)))
    return 2.0 * t * poly + e.astype(jnp.float32) * LN2


def _make_sc_call():
    info = plsc.get_sparse_core_info()
    assert info.num_cores * info.num_subcores == 32, (
        "kernel assumes 2 SparseCores x 16 vector subcores")
    mesh = plsc.VectorSubcoreMesh(core_axis_name="c", subcore_axis_name="s")

    @functools.partial(
        pl.kernel,
        mesh=mesh,
        out_type=jax.ShapeDtypeStruct((LPAD, ROWS), jnp.float32),
        scratch_types=[
            pltpu.VMEM((LPAD,), jnp.int32),
            pltpu.VMEM((LBLK, RBLK), jnp.float32),
            pltpu.VMEM((2, RBLK), jnp.float32),
            pltpu.VMEM((NQ, 2, RBLK), jnp.float32),
            pltpu.VMEM_SHARED((16, 2, RBLK), jnp.float32),
            pltpu.SemaphoreType.DMA,
        ],
    )
    def sc_call(lt_hbm, w_hbm, out_hbm, w_v, vals_v,
                pstat_v, peers_v, stats_sp, sem_in):
        c = lax.axis_index("c")
        s = lax.axis_index("s")
        b = c * 4 + s // 4
        q = s % 4
        row0 = pl.multiple_of(b * RBLK, RBLK)
        lab0 = q * LBLK

        pltpu.sync_copy(w_hbm, w_v.at[pl.ds(0, NLAB)])
        wj = []
        for cc in range(LBLK // 16):
            wc = w_v[pl.ds(lab0 + 16 * cc, 16)]
            for k in range(16):
                slot = lab0 + 16 * cc + k
                wj.append(jnp.where(slot < NLAB, wc[k], 0))

        for j in range(LBLK):
            pltpu.async_copy(
                lt_hbm.at[wj[j], pl.ds(row0, RBLK)],
                vals_v.at[j], sem_in)
        pltpu.make_async_copy(
            lt_hbm.at[pl.ds(0, LBLK), pl.ds(0, RBLK)], vals_v, sem_in).wait()

        def p1_body(j, maxacc):
            valid = (lab0 + j) < NLAB
            out = []
            for h in range(NH):
                xl = vals_v[j, pl.ds(16 * h, 16)]
                out.append(jnp.maximum(maxacc[h], jnp.where(valid, xl, NEG)))
            return tuple(out)

        maxacc = lax.fori_loop(
            0, LBLK, p1_body,
            tuple(jnp.full((16,), NEG, jnp.float32) for _ in range(NH)))
        for h in range(NH):
            pstat_v[0, pl.ds(16 * h, 16)] = maxacc[h]

        def p2_body(j, sumacc):
            valid = (lab0 + j) < NLAB
            out = []
            for h in range(NH):
                e = jnp.exp(vals_v[j, pl.ds(16 * h, 16)] - maxacc[h])
                out.append(sumacc[h] + jnp.where(valid, e, 0.0))
            return tuple(out)

        sumacc = lax.fori_loop(
            0, LBLK, p2_body,
            tuple(jnp.zeros((16,), jnp.float32) for _ in range(NH)))
        for h in range(NH):
            pstat_v[1, pl.ds(16 * h, 16)] = sumacc[h]

        pltpu.sync_copy(pstat_v, stats_sp.at[s])
        plsc.subcore_barrier()
        s0 = (s // 4) * 4
        pltpu.sync_copy(stats_sp.at[pl.ds(s0, NQ)], peers_v)

        def comb_body(h, carry):
            pm = [peers_v[r, 0, pl.ds(16 * h, 16)] for r in range(NQ)]
            m = pm[0]
            for r in range(1, NQ):
                m = jnp.maximum(m, pm[r])
            ssum = jnp.zeros((16,), jnp.float32)
            for r in range(NQ):
                ssum = ssum + (peers_v[r, 1, pl.ds(16 * h, 16)]
                               * jnp.exp(pm[r] - m))
            pstat_v[0, pl.ds(16 * h, 16)] = _log_lanes(ssum) + m
            return carry

        lax.fori_loop(0, NH, comb_body, 0)
        logz = [pstat_v[0, pl.ds(16 * h, 16)] for h in range(NH)]

        def p3_body(j, carry):
            for h in range(NH):
                vals_v[j, pl.ds(16 * h, 16)] = (
                    vals_v[j, pl.ds(16 * h, 16)] - logz[h])
            return carry

        lax.fori_loop(0, LBLK, p3_body, 0)
        pltpu.sync_copy(
            vals_v,
            out_hbm.at[pl.ds(pl.multiple_of(lab0, LBLK), LBLK),
                       pl.ds(row0, RBLK)])

    return sc_call


_SC_CALL = _make_sc_call()


def kernel(logits, word2label):
    lt = logits.T
    w = word2label.astype(jnp.int32)
    out_cm = _SC_CALL(lt, w)
    return out_cm[:NLAB].T
